# baseline (speedup 1.0000x reference)
import jax
import jax.numpy as jnp
from jax import lax
from jax.experimental import pallas as pl
from jax.experimental.pallas import tpu as pltpu
from jax.experimental.pallas import tpu_sc as plsc

T = 1024
HID = 2048
INTER = 1408
E = 16
GSZ = 4
RSF = 2.5
SH_INTER = 2816
SH_TILE = 256
NA = 2048
B = 128
NB = 32
NBB = NB * B
NEG = -1e9

_HI = lax.Precision.HIGHEST


def _dot(a, b):
    return lax.dot_general(a, b, (((1,), (0,)), ((), ())),
                           precision=_HI, preferred_element_type=jnp.float32)


def _router_body(x_ref, wg_ref, b_ref, pos_ref, w_ref, be_ref):
    f32 = jnp.float32
    x = x_ref[...]
    logits = lax.dot_general(x, wg_ref[...], (((1,), (0,)), ((), ())),
                             preferred_element_type=jnp.float32)
    scores = jax.nn.sigmoid(logits)
    s4 = scores + b_ref[...]

    r16 = lax.broadcasted_iota(jnp.int32, (E, E), 0)
    c16 = lax.broadcasted_iota(jnp.int32, (E, E), 1)

    def roll(v, k):
        tgt = (c16 // GSZ) * GSZ + ((c16 % GSZ + k) % GSZ)
        return _dot(v, (r16 == tgt).astype(f32))

    u = jnp.maximum(s4 + roll(s4, 1), s4 + roll(s4, 2))
    v = jnp.maximum(u, roll(u, 1))
    gsc = jnp.maximum(v, roll(v, 2))

    lane = lax.broadcasted_iota(jnp.int32, (T, E), 1)
    gid = lane // GSZ
    gmax = jnp.max(gsc, axis=1, keepdims=True)
    i1g = jnp.min(jnp.where(gsc == gmax, gid, 99), axis=1, keepdims=True)
    g2s = jnp.where(gid == i1g, jnp.float32(-1e30), gsc)
    g2max = jnp.max(g2s, axis=1, keepdims=True)
    i2g = jnp.min(jnp.where(g2s == g2max, gid, 99), axis=1, keepdims=True)
    gmask = (gid == i1g) | (gid == i2g)

    msk = jnp.where(gmask, s4, NEG)
    m1 = jnp.max(msk, axis=1, keepdims=True)
    e1 = jnp.min(jnp.where(msk == m1, lane, 99), axis=1, keepdims=True)
    msk2 = jnp.where(lane == e1, NEG, msk)
    m2 = jnp.max(msk2, axis=1, keepdims=True)
    e2 = jnp.min(jnp.where(msk2 == m2, lane, 99), axis=1, keepdims=True)

    w1 = jnp.sum(jnp.where(lane == e1, scores, 0.0), axis=1, keepdims=True)
    w2 = jnp.sum(jnp.where(lane == e2, scores, 0.0), axis=1, keepdims=True)
    ws = w1 + w2 + 1e-20
    w1 = w1 / ws * RSF
    w2 = w2 / ws * RSF

    ecol = jnp.concatenate([e1, e2], axis=0)
    wcol = jnp.concatenate([w1, w2], axis=0)

    lane16 = lax.broadcasted_iota(jnp.int32, (NA, E), 1)
    oh = (ecol == lane16).astype(f32)
    cum = oh
    sh = 1
    while sh < NA:
        cum = cum + jnp.concatenate(
            [jnp.zeros((sh, E), f32), cum[:NA - sh, :]], axis=0)
        sh *= 2
    rank = jnp.sum(cum * oh, axis=1, keepdims=True) - 1.0
    cnt = cum[NA - 1:NA, :]
    cpad = ((cnt.astype(jnp.int32) + (B - 1)) // B) * B
    cpad_f = cpad.astype(f32)
    off = _dot(cpad_f, (r16 < c16).astype(f32))
    offend = off + cpad_f
    off_e = jnp.sum(oh * off, axis=1, keepdims=True)
    pos_ref[...] = (off_e + rank).astype(jnp.int32)
    w_ref[...] = wcol

    brow = lax.broadcasted_iota(jnp.int32, (NB, E), 0) * B
    be = jnp.sum((brow >= offend.astype(jnp.int32)).astype(jnp.int32),
                 axis=1, keepdims=True)
    be_ref[...] = jnp.minimum(be, E - 1)


def _router(x, w_gate, bias2d):
    return pl.pallas_call(
        _router_body,
        out_shape=[jax.ShapeDtypeStruct((NA, 1), jnp.int32),
                   jax.ShapeDtypeStruct((NA, 1), jnp.float32),
                   jax.ShapeDtypeStruct((NB, 1), jnp.int32)],
    )(x, w_gate, bias2d)


def _dispatch_body(pos_hbm, w_hbm, rows_hbm, wslot_hbm, posv, wv, rowsv, wsv):
    cid = lax.axis_index("c")
    sid = lax.axis_index("s")

    @pl.when(jnp.logical_and(cid == 0, sid == 0))
    def _():
        pltpu.sync_copy(pos_hbm, posv)
        pltpu.sync_copy(w_hbm, wv)
        zi = jnp.zeros((16,), jnp.int32)
        zf = jnp.zeros((16,), jnp.float32)

        def zbody(i, carry):
            rowsv[pl.ds(i * 16, 16)] = zi
            wsv[pl.ds(i * 16, 16)] = zf
            return carry

        lax.fori_loop(0, NBB // 16, zbody, 0)
        lanes = lax.iota(jnp.int32, 16)

        def sbody(i, carry):
            idx = posv[pl.ds(i * 16, 16)]
            tok = jnp.bitwise_and(i * 16 + lanes, T - 1)
            plsc.store_scatter(rowsv, [idx], tok)
            plsc.store_scatter(wsv, [idx], wv[pl.ds(i * 16, 16)])
            return carry

        lax.fori_loop(0, NA // 16, sbody, 0)
        pltpu.sync_copy(rowsv, rows_hbm)
        pltpu.sync_copy(wsv, wslot_hbm)


def _dispatch(posf, wf):
    fn = pl.kernel(
        _dispatch_body,
        out_type=[jax.ShapeDtypeStruct((NBB,), jnp.int32),
                  jax.ShapeDtypeStruct((NBB,), jnp.float32)],
        mesh=plsc.VectorSubcoreMesh(core_axis_name="c", subcore_axis_name="s"),
        scratch_types=[pltpu.VMEM((NA,), jnp.int32),
                       pltpu.VMEM((NA,), jnp.float32),
                       pltpu.VMEM((NBB,), jnp.int32),
                       pltpu.VMEM((NBB,), jnp.float32)],
        compiler_params=pltpu.CompilerParams(needs_layout_passes=False),
    )
    return fn(posf, wf)


def _gather_body(x_hbm, rows_hbm, xg_hbm, idxc, buf, sem):
    cid = lax.axis_index("c")
    sid = lax.axis_index("s")
    wid = sid * 2 + cid
    base = wid * (NBB // 32)

    def body(j, carry):
        pltpu.sync_copy(rows_hbm.at[pl.ds(base + j * 32, 32)], idxc)
        pltpu.async_copy(x_hbm.at[idxc], buf, sem).wait()
        pltpu.sync_copy(buf, xg_hbm.at[pl.ds(base + j * 32, 32)])
        return carry

    lax.fori_loop(0, (NBB // 32) // 32, body, 0)


def _gather(x, rows):
    fn = pl.kernel(
        _gather_body,
        out_type=jax.ShapeDtypeStruct((NBB, HID), jnp.float32),
        mesh=plsc.VectorSubcoreMesh(core_axis_name="c", subcore_axis_name="s"),
        scratch_types=[pltpu.VMEM((32,), jnp.int32),
                       pltpu.VMEM((32, HID), jnp.float32),
                       pltpu.SemaphoreType.DMA],
    )
    return fn(x, rows)


def _shared_body(x_ref, wsg_ref, wsu_ref, wsd_ref, out_ref):
    i = pl.program_id(0)
    x = x_ref[...]
    hg = _dot(x, wsg_ref[...])
    hu = _dot(x, wsu_ref[...])
    part = _dot(hg * jax.nn.sigmoid(hg) * hu, wsd_ref[...])

    @pl.when(i == 0)
    def _():
        out_ref[...] = part

    @pl.when(i > 0)
    def _():
        out_ref[...] = out_ref[...] + part


def _shared(x, wsg, wsu, wsd):
    nst = SH_INTER // SH_TILE
    return pl.pallas_call(
        _shared_body,
        grid=(nst,),
        in_specs=[
            pl.BlockSpec((T, HID), lambda i: (0, 0)),
            pl.BlockSpec((HID, SH_TILE), lambda i: (0, i)),
            pl.BlockSpec((HID, SH_TILE), lambda i: (0, i)),
            pl.BlockSpec((SH_TILE, HID), lambda i: (i, 0)),
        ],
        out_specs=pl.BlockSpec((T, HID), lambda i: (0, 0)),
        out_shape=jax.ShapeDtypeStruct((T, HID), jnp.float32),
        compiler_params=pltpu.CompilerParams(
            dimension_semantics=("arbitrary",)),
    )(x, wsg, wsu, wsd)


def _moe_up_body(be_ref, xg_ref, wg_ref, wu_ref, ws_ref, h_ref):
    xb = xg_ref[0]
    hg = _dot(xb, wg_ref[0])
    hu = _dot(xb, wu_ref[0])
    h_ref[0] = hg * jax.nn.sigmoid(hg) * hu * ws_ref[0]


def _moe_up(bef, xg3, wg, wu, ws3):
    grid_spec = pltpu.PrefetchScalarGridSpec(
        num_scalar_prefetch=1,
        grid=(NB,),
        in_specs=[
            pl.BlockSpec((1, B, HID), lambda b, be: (b, 0, 0)),
            pl.BlockSpec((1, HID, INTER), lambda b, be: (be[b], 0, 0)),
            pl.BlockSpec((1, HID, INTER), lambda b, be: (be[b], 0, 0)),
            pl.BlockSpec((1, B, 1), lambda b, be: (b, 0, 0)),
        ],
        out_specs=pl.BlockSpec((1, B, INTER), lambda b, be: (b, 0, 0)),
    )
    return pl.pallas_call(
        _moe_up_body,
        grid_spec=grid_spec,
        out_shape=jax.ShapeDtypeStruct((NB, B, INTER), jnp.float32),
        compiler_params=pltpu.CompilerParams(
            dimension_semantics=("arbitrary",),
            vmem_limit_bytes=60 * 1024 * 1024),
    )(bef, xg3, wg, wu, ws3)


def _moe_down_body(be_ref, h_ref, wd_ref, out_ref):
    out_ref[0] = _dot(h_ref[0], wd_ref[0])


def _moe_down(bef, h3, wd):
    grid_spec = pltpu.PrefetchScalarGridSpec(
        num_scalar_prefetch=1,
        grid=(NB,),
        in_specs=[
            pl.BlockSpec((1, B, INTER), lambda b, be: (b, 0, 0)),
            pl.BlockSpec((1, INTER, HID), lambda b, be: (be[b], 0, 0)),
        ],
        out_specs=pl.BlockSpec((1, B, HID), lambda b, be: (b, 0, 0)),
    )
    return pl.pallas_call(
        _moe_down_body,
        grid_spec=grid_spec,
        out_shape=jax.ShapeDtypeStruct((NB, B, HID), jnp.float32),
        compiler_params=pltpu.CompilerParams(
            dimension_semantics=("arbitrary",),
            vmem_limit_bytes=60 * 1024 * 1024),
    )(bef, h3, wd)


def _combine_body(yg_hbm, pos_hbm, sh_hbm, out_hbm, p0v, p1v, b0, b1, bs, sem):
    cid = lax.axis_index("c")
    sid = lax.axis_index("s")
    wid = sid * 2 + cid
    tbase = wid * (T // 32)

    def body(j, carry):
        tb = tbase + j * 8
        pltpu.sync_copy(pos_hbm.at[pl.ds(tb, 8)], p0v)
        pltpu.sync_copy(pos_hbm.at[pl.ds(T + tb, 8)], p1v)
        pltpu.async_copy(yg_hbm.at[p0v], b0, sem).wait()
        pltpu.async_copy(yg_hbm.at[p1v], b1, sem).wait()
        pltpu.sync_copy(sh_hbm.at[pl.ds(tb, 8)], bs)
        for r in range(8):
            def cadd(k, carry2, r=r):
                sl = pl.ds(k * 16, 16)
                b0[r, sl] = b0[r, sl] + b1[r, sl] + bs[r, sl]
                return carry2
            lax.fori_loop(0, HID // 16, cadd, 0)
        pltpu.sync_copy(b0, out_hbm.at[pl.ds(tb, 8)])
        return carry

    lax.fori_loop(0, (T // 32) // 8, body, 0)


def _combine(yg, posf, shared):
    fn = pl.kernel(
        _combine_body,
        out_type=jax.ShapeDtypeStruct((T, HID), jnp.float32),
        mesh=plsc.VectorSubcoreMesh(core_axis_name="c", subcore_axis_name="s"),
        scratch_types=[pltpu.VMEM((8,), jnp.int32),
                       pltpu.VMEM((8,), jnp.int32),
                       pltpu.VMEM((8, HID), jnp.float32),
                       pltpu.VMEM((8, HID), jnp.float32),
                       pltpu.VMEM((8, HID), jnp.float32),
                       pltpu.SemaphoreType.DMA],
    )
    return fn(yg, posf, shared)


def kernel(hidden_states, residual, layer_id, W_gate, bias_corr,
           Wg, Wu, Wd, Wsg, Wsu, Wsd):
    x = hidden_states
    pos, wcol, be = _router(x, W_gate, bias_corr.reshape(1, E))
    posf = pos.reshape(NA)
    wf = wcol.reshape(NA)
    bef = be.reshape(NB)
    rows, wslot = _dispatch(posf, wf)
    xg = _gather(x, rows)
    shared = _shared(x, Wsg, Wsu, Wsd)
    h3 = _moe_up(bef, xg.reshape(NB, B, HID), Wg, Wu,
                 wslot.reshape(NB, B, 1))
    yg = _moe_down(bef, h3, Wd)
    return _combine(yg.reshape(NBB, HID), posf, shared)


# speedup vs baseline: 1.5680x; 1.5680x over previous
import jax
import jax.numpy as jnp
from jax import lax
from jax.experimental import pallas as pl
from jax.experimental.pallas import tpu as pltpu
from jax.experimental.pallas import tpu_sc as plsc

T = 1024
HID = 2048
INTER = 1408
E = 16
GSZ = 4
RSF = 2.5
SH_INTER = 2816
SH_TILE = 256
NA = 2048
B = 128
NB = 32
NBB = NB * B
NEG = -1e9

_HI = lax.Precision.HIGHEST


def _dot(a, b):
    return lax.dot_general(a, b, (((1,), (0,)), ((), ())),
                           precision=_HI, preferred_element_type=jnp.float32)


def _dotd(a, b):
    return lax.dot_general(a, b, (((1,), (0,)), ((), ())),
                           preferred_element_type=jnp.float32)


def _router_body(x_ref, wg_ref, b_ref, pos_ref, w_ref, be_ref):
    f32 = jnp.float32
    x = x_ref[...]
    logits = lax.dot_general(x, wg_ref[...], (((1,), (0,)), ((), ())),
                             preferred_element_type=jnp.float32)
    scores = jax.nn.sigmoid(logits)
    s4 = scores + b_ref[...]

    r16 = lax.broadcasted_iota(jnp.int32, (E, E), 0)
    c16 = lax.broadcasted_iota(jnp.int32, (E, E), 1)

    def roll(v, k):
        tgt = (c16 // GSZ) * GSZ + ((c16 % GSZ + k) % GSZ)
        return _dot(v, (r16 == tgt).astype(f32))

    u = jnp.maximum(s4 + roll(s4, 1), s4 + roll(s4, 2))
    v = jnp.maximum(u, roll(u, 1))
    gsc = jnp.maximum(v, roll(v, 2))

    lane = lax.broadcasted_iota(jnp.int32, (T, E), 1)
    gid = lane // GSZ
    gmax = jnp.max(gsc, axis=1, keepdims=True)
    i1g = jnp.min(jnp.where(gsc == gmax, gid, 99), axis=1, keepdims=True)
    g2s = jnp.where(gid == i1g, jnp.float32(-1e30), gsc)
    g2max = jnp.max(g2s, axis=1, keepdims=True)
    i2g = jnp.min(jnp.where(g2s == g2max, gid, 99), axis=1, keepdims=True)
    gmask = (gid == i1g) | (gid == i2g)

    msk = jnp.where(gmask, s4, NEG)
    m1 = jnp.max(msk, axis=1, keepdims=True)
    e1 = jnp.min(jnp.where(msk == m1, lane, 99), axis=1, keepdims=True)
    msk2 = jnp.where(lane == e1, NEG, msk)
    m2 = jnp.max(msk2, axis=1, keepdims=True)
    e2 = jnp.min(jnp.where(msk2 == m2, lane, 99), axis=1, keepdims=True)

    w1 = jnp.sum(jnp.where(lane == e1, scores, 0.0), axis=1, keepdims=True)
    w2 = jnp.sum(jnp.where(lane == e2, scores, 0.0), axis=1, keepdims=True)
    ws = w1 + w2 + 1e-20
    w1 = w1 / ws * RSF
    w2 = w2 / ws * RSF

    ecol = jnp.concatenate([e1, e2], axis=0)
    wcol = jnp.concatenate([w1, w2], axis=0)

    lane16 = lax.broadcasted_iota(jnp.int32, (NA, E), 1)
    oh = (ecol == lane16).astype(f32)
    cum = oh
    sh = 1
    while sh < NA:
        cum = cum + jnp.concatenate(
            [jnp.zeros((sh, E), f32), cum[:NA - sh, :]], axis=0)
        sh *= 2
    rank = jnp.sum(cum * oh, axis=1, keepdims=True) - 1.0
    cnt = cum[NA - 1:NA, :]
    cpad = ((cnt.astype(jnp.int32) + (B - 1)) // B) * B
    cpad_f = cpad.astype(f32)
    off = _dot(cpad_f, (r16 < c16).astype(f32))
    offend = off + cpad_f
    off_e = jnp.sum(oh * off, axis=1, keepdims=True)
    pos_ref[...] = (off_e + rank).astype(jnp.int32)
    w_ref[...] = wcol

    brow = lax.broadcasted_iota(jnp.int32, (NB, E), 0) * B
    be = jnp.sum((brow >= offend.astype(jnp.int32)).astype(jnp.int32),
                 axis=1, keepdims=True)
    be_ref[...] = jnp.minimum(be, E - 1)


def _router(x, w_gate, bias2d):
    return pl.pallas_call(
        _router_body,
        out_shape=[jax.ShapeDtypeStruct((NA, 1), jnp.int32),
                   jax.ShapeDtypeStruct((NA, 1), jnp.float32),
                   jax.ShapeDtypeStruct((NB, 1), jnp.int32)],
    )(x, w_gate, bias2d)


def _dispatch_body(pos_hbm, w_hbm, rows_hbm, wslot_hbm, posv, wv, rowsv, wsv):
    cid = lax.axis_index("c")
    sid = lax.axis_index("s")

    @pl.when(jnp.logical_and(cid == 0, sid == 0))
    def _():
        pltpu.sync_copy(pos_hbm, posv)
        pltpu.sync_copy(w_hbm, wv)
        zi = jnp.zeros((16,), jnp.int32)
        zf = jnp.zeros((16,), jnp.float32)

        def zbody(i, carry):
            rowsv[pl.ds(i * 16, 16)] = zi
            wsv[pl.ds(i * 16, 16)] = zf
            return carry

        lax.fori_loop(0, NBB // 16, zbody, 0)
        lanes = lax.iota(jnp.int32, 16)

        def sbody(i, carry):
            idx = posv[pl.ds(i * 16, 16)]
            tok = jnp.bitwise_and(i * 16 + lanes, T - 1)
            plsc.store_scatter(rowsv, [idx], tok)
            plsc.store_scatter(wsv, [idx], wv[pl.ds(i * 16, 16)])
            return carry

        lax.fori_loop(0, NA // 16, sbody, 0)
        pltpu.sync_copy(rowsv, rows_hbm)
        pltpu.sync_copy(wsv, wslot_hbm)


def _dispatch(posf, wf):
    fn = pl.kernel(
        _dispatch_body,
        out_type=[jax.ShapeDtypeStruct((NBB,), jnp.int32),
                  jax.ShapeDtypeStruct((NBB,), jnp.float32)],
        mesh=plsc.VectorSubcoreMesh(core_axis_name="c", subcore_axis_name="s"),
        scratch_types=[pltpu.VMEM((NA,), jnp.int32),
                       pltpu.VMEM((NA,), jnp.float32),
                       pltpu.VMEM((NBB,), jnp.int32),
                       pltpu.VMEM((NBB,), jnp.float32)],
        compiler_params=pltpu.CompilerParams(needs_layout_passes=False),
    )
    return fn(posf, wf)


_GCH = 16
_GN = (NBB // 32) // _GCH


def _gather_body(x_hbm, rows_hbm, xg_hbm, idxv, b0, b1, b2,
                 g0, g1, g2, o0, o1, o2):
    cid = lax.axis_index("c")
    sid = lax.axis_index("s")
    wid = sid * 2 + cid
    base = wid * (NBB // 32)
    bufs = (b0, b1, b2)
    gsems = (g0, g1, g2)
    osems = (o0, o1, o2)

    pltpu.sync_copy(rows_hbm.at[pl.ds(base, NBB // 32)], idxv)

    def gstart(j):
        return pltpu.async_copy(
            x_hbm.at[idxv.at[pl.ds(j * _GCH, _GCH)]], bufs[j % 3],
            gsems[j % 3])

    def ostart(j):
        return pltpu.async_copy(
            bufs[j % 3], xg_hbm.at[pl.ds(base + j * _GCH, _GCH)],
            osems[j % 3])

    gd = {0: gstart(0), 1: gstart(1)}
    od = {}
    for j in range(_GN):
        if j + 2 < _GN:
            if j - 1 >= 0:
                od[j - 1].wait()
            gd[j + 2] = gstart(j + 2)
        gd[j].wait()
        od[j] = ostart(j)
    od[_GN - 2].wait()
    od[_GN - 1].wait()


def _gather(x, rows):
    fn = pl.kernel(
        _gather_body,
        out_type=jax.ShapeDtypeStruct((NBB, HID), jnp.float32),
        mesh=plsc.VectorSubcoreMesh(core_axis_name="c", subcore_axis_name="s"),
        scratch_types=[pltpu.VMEM((NBB // 32,), jnp.int32),
                       pltpu.VMEM((_GCH, HID), jnp.float32),
                       pltpu.VMEM((_GCH, HID), jnp.float32),
                       pltpu.VMEM((_GCH, HID), jnp.float32),
                       pltpu.SemaphoreType.DMA, pltpu.SemaphoreType.DMA,
                       pltpu.SemaphoreType.DMA, pltpu.SemaphoreType.DMA,
                       pltpu.SemaphoreType.DMA, pltpu.SemaphoreType.DMA],
    )
    return fn(x, rows)


def _shared_body(x_ref, wsg_ref, wsu_ref, wsd_ref, out_ref):
    i = pl.program_id(0)
    x = x_ref[...]
    hg = _dotd(x, wsg_ref[...])
    hu = _dotd(x, wsu_ref[...])
    part = _dotd(hg * jax.nn.sigmoid(hg) * hu, wsd_ref[...])

    @pl.when(i == 0)
    def _():
        out_ref[...] = part

    @pl.when(i > 0)
    def _():
        out_ref[...] = out_ref[...] + part


def _shared(x, wsg, wsu, wsd):
    nst = SH_INTER // SH_TILE
    return pl.pallas_call(
        _shared_body,
        grid=(nst,),
        in_specs=[
            pl.BlockSpec((T, HID), lambda i: (0, 0)),
            pl.BlockSpec((HID, SH_TILE), lambda i: (0, i)),
            pl.BlockSpec((HID, SH_TILE), lambda i: (0, i)),
            pl.BlockSpec((SH_TILE, HID), lambda i: (i, 0)),
        ],
        out_specs=pl.BlockSpec((T, HID), lambda i: (0, 0)),
        out_shape=jax.ShapeDtypeStruct((T, HID), jnp.float32),
        compiler_params=pltpu.CompilerParams(
            dimension_semantics=("arbitrary",)),
    )(x, wsg, wsu, wsd)


def _moe_up_body(be_ref, xg_ref, wg_ref, wu_ref, ws_ref, h_ref):
    xb = xg_ref[0]
    hg = _dotd(xb, wg_ref[0])
    hu = _dotd(xb, wu_ref[0])
    h_ref[0] = hg * jax.nn.sigmoid(hg) * hu * ws_ref[0]


def _moe_up(bef, xg3, wg, wu, ws3):
    grid_spec = pltpu.PrefetchScalarGridSpec(
        num_scalar_prefetch=1,
        grid=(NB,),
        in_specs=[
            pl.BlockSpec((1, B, HID), lambda b, be: (b, 0, 0)),
            pl.BlockSpec((1, HID, INTER), lambda b, be: (be[b], 0, 0)),
            pl.BlockSpec((1, HID, INTER), lambda b, be: (be[b], 0, 0)),
            pl.BlockSpec((1, B, 1), lambda b, be: (b, 0, 0)),
        ],
        out_specs=pl.BlockSpec((1, B, INTER), lambda b, be: (b, 0, 0)),
    )
    return pl.pallas_call(
        _moe_up_body,
        grid_spec=grid_spec,
        out_shape=jax.ShapeDtypeStruct((NB, B, INTER), jnp.float32),
        compiler_params=pltpu.CompilerParams(
            dimension_semantics=("arbitrary",),
            vmem_limit_bytes=60 * 1024 * 1024),
    )(bef, xg3, wg, wu, ws3)


def _moe_down_body(be_ref, h_ref, wd_ref, out_ref):
    out_ref[0] = _dotd(h_ref[0], wd_ref[0])


def _moe_down(bef, h3, wd):
    grid_spec = pltpu.PrefetchScalarGridSpec(
        num_scalar_prefetch=1,
        grid=(NB,),
        in_specs=[
            pl.BlockSpec((1, B, INTER), lambda b, be: (b, 0, 0)),
            pl.BlockSpec((1, INTER, HID), lambda b, be: (be[b], 0, 0)),
        ],
        out_specs=pl.BlockSpec((1, B, HID), lambda b, be: (b, 0, 0)),
    )
    return pl.pallas_call(
        _moe_down_body,
        grid_spec=grid_spec,
        out_shape=jax.ShapeDtypeStruct((NB, B, HID), jnp.float32),
        compiler_params=pltpu.CompilerParams(
            dimension_semantics=("arbitrary",),
            vmem_limit_bytes=60 * 1024 * 1024),
    )(bef, h3, wd)


_CCH = 8
_CN = (T // 32) // _CCH


def _combine_body(yg_hbm, pos_hbm, sh_hbm, out_hbm, p0v, p1v,
                  a0, a1, c0, c1, s0, s1, gs0, gs1, os0, os1):
    cid = lax.axis_index("c")
    sid = lax.axis_index("s")
    wid = sid * 2 + cid
    tbase = wid * (T // 32)
    abufs = (a0, a1)
    bbufs = (c0, c1)
    sbufs = (s0, s1)
    gsems = (gs0, gs1)
    osems = (os0, os1)

    pltpu.sync_copy(pos_hbm.at[pl.ds(tbase, T // 32)], p0v)
    pltpu.sync_copy(pos_hbm.at[pl.ds(T + tbase, T // 32)], p1v)

    def gstart(j):
        s = j % 2
        da = pltpu.async_copy(yg_hbm.at[p0v.at[pl.ds(j * _CCH, _CCH)]],
                              abufs[s], gsems[s])
        db = pltpu.async_copy(yg_hbm.at[p1v.at[pl.ds(j * _CCH, _CCH)]],
                              bbufs[s], gsems[s])
        dc = pltpu.async_copy(sh_hbm.at[pl.ds(tbase + j * _CCH, _CCH)],
                              sbufs[s], gsems[s])
        return (da, db, dc)

    gd = {0: gstart(0)}
    od = {}
    for j in range(_CN):
        if j + 1 < _CN:
            if j - 1 >= 0:
                od[j - 1].wait()
            gd[j + 1] = gstart(j + 1)
        for d in gd[j]:
            d.wait()
        s = j % 2
        a, c, sb = abufs[s], bbufs[s], sbufs[s]
        for r in range(_CCH):
            def cadd(k, carry2, r=r, a=a, c=c, sb=sb):
                sl = pl.ds(k * 16, 16)
                a[r, sl] = a[r, sl] + c[r, sl] + sb[r, sl]
                return carry2
            lax.fori_loop(0, HID // 16, cadd, 0)
        od[j] = pltpu.async_copy(
            a, out_hbm.at[pl.ds(tbase + j * _CCH, _CCH)], osems[s])
    od[_CN - 2].wait()
    od[_CN - 1].wait()


def _combine(yg, posf, shared):
    fn = pl.kernel(
        _combine_body,
        out_type=jax.ShapeDtypeStruct((T, HID), jnp.float32),
        mesh=plsc.VectorSubcoreMesh(core_axis_name="c", subcore_axis_name="s"),
        scratch_types=[pltpu.VMEM((T // 32,), jnp.int32),
                       pltpu.VMEM((T // 32,), jnp.int32),
                       pltpu.VMEM((_CCH, HID), jnp.float32),
                       pltpu.VMEM((_CCH, HID), jnp.float32),
                       pltpu.VMEM((_CCH, HID), jnp.float32),
                       pltpu.VMEM((_CCH, HID), jnp.float32),
                       pltpu.VMEM((_CCH, HID), jnp.float32),
                       pltpu.VMEM((_CCH, HID), jnp.float32),
                       pltpu.SemaphoreType.DMA, pltpu.SemaphoreType.DMA,
                       pltpu.SemaphoreType.DMA, pltpu.SemaphoreType.DMA],
    )
    return fn(yg, posf, shared)


def kernel(hidden_states, residual, layer_id, W_gate, bias_corr,
           Wg, Wu, Wd, Wsg, Wsu, Wsd):
    x = hidden_states
    pos, wcol, be = _router(x, W_gate, bias_corr.reshape(1, E))
    posf = pos.reshape(NA)
    wf = wcol.reshape(NA)
    bef = be.reshape(NB)
    rows, wslot = _dispatch(posf, wf)
    xg = _gather(x, rows)
    shared = _shared(x, Wsg, Wsu, Wsd)
    h3 = _moe_up(bef, xg.reshape(NB, B, HID), Wg, Wu,
                 wslot.reshape(NB, B, 1))
    yg = _moe_down(bef, h3, Wd)
    return _combine(yg.reshape(NBB, HID), posf, shared)


# speedup vs baseline: 2.2866x; 1.4583x over previous
import jax
import jax.numpy as jnp
from jax import lax
from jax.experimental import pallas as pl
from jax.experimental.pallas import tpu as pltpu
from jax.experimental.pallas import tpu_sc as plsc

T = 1024
HID = 2048
INTER = 1408
E = 16
GSZ = 4
RSF = 2.5
SH_INTER = 2816
SH_TILE = 256
NA = 2048
B = 128
NB = 32
NBB = NB * B
NEG = -1e9

_HI = lax.Precision.HIGHEST


def _dot(a, b):
    return lax.dot_general(a, b, (((1,), (0,)), ((), ())),
                           precision=_HI, preferred_element_type=jnp.float32)


def _dotd(a, b):
    return lax.dot_general(a, b, (((1,), (0,)), ((), ())),
                           preferred_element_type=jnp.float32)


def _router_body(x_ref, wg_ref, b_ref, pos_ref, w_ref, be_ref, xbf_ref):
    f32 = jnp.float32
    x = x_ref[...]
    logits = lax.dot_general(x, wg_ref[...], (((1,), (0,)), ((), ())),
                             preferred_element_type=jnp.float32)
    scores = jax.nn.sigmoid(logits)
    s4 = scores + b_ref[...]

    r16 = lax.broadcasted_iota(jnp.int32, (E, E), 0)
    c16 = lax.broadcasted_iota(jnp.int32, (E, E), 1)

    def roll(v, k):
        tgt = (c16 // GSZ) * GSZ + ((c16 % GSZ + k) % GSZ)
        return _dot(v, (r16 == tgt).astype(f32))

    u = jnp.maximum(s4 + roll(s4, 1), s4 + roll(s4, 2))
    v = jnp.maximum(u, roll(u, 1))
    gsc = jnp.maximum(v, roll(v, 2))

    lane = lax.broadcasted_iota(jnp.int32, (T, E), 1)
    gid = lane // GSZ
    gmax = jnp.max(gsc, axis=1, keepdims=True)
    i1g = jnp.min(jnp.where(gsc == gmax, gid, 99), axis=1, keepdims=True)
    g2s = jnp.where(gid == i1g, jnp.float32(-1e30), gsc)
    g2max = jnp.max(g2s, axis=1, keepdims=True)
    i2g = jnp.min(jnp.where(g2s == g2max, gid, 99), axis=1, keepdims=True)
    gmask = (gid == i1g) | (gid == i2g)

    msk = jnp.where(gmask, s4, NEG)
    m1 = jnp.max(msk, axis=1, keepdims=True)
    e1 = jnp.min(jnp.where(msk == m1, lane, 99), axis=1, keepdims=True)
    msk2 = jnp.where(lane == e1, NEG, msk)
    m2 = jnp.max(msk2, axis=1, keepdims=True)
    e2 = jnp.min(jnp.where(msk2 == m2, lane, 99), axis=1, keepdims=True)

    w1 = jnp.sum(jnp.where(lane == e1, scores, 0.0), axis=1, keepdims=True)
    w2 = jnp.sum(jnp.where(lane == e2, scores, 0.0), axis=1, keepdims=True)
    ws = w1 + w2 + 1e-20
    w1 = w1 / ws * RSF
    w2 = w2 / ws * RSF

    ecol = jnp.concatenate([e1, e2], axis=0)
    wcol = jnp.concatenate([w1, w2], axis=0)

    lane16 = lax.broadcasted_iota(jnp.int32, (NA, E), 1)
    oh = (ecol == lane16).astype(f32)
    cum = oh
    sh = 1
    while sh < NA:
        cum = cum + jnp.concatenate(
            [jnp.zeros((sh, E), f32), cum[:NA - sh, :]], axis=0)
        sh *= 2
    rank = jnp.sum(cum * oh, axis=1, keepdims=True) - 1.0
    cnt = cum[NA - 1:NA, :]
    cpad = ((cnt.astype(jnp.int32) + (B - 1)) // B) * B
    cpad_f = cpad.astype(f32)
    off = _dot(cpad_f, (r16 < c16).astype(f32))
    offend = off + cpad_f
    off_e = jnp.sum(oh * off, axis=1, keepdims=True)
    pos_ref[...] = (off_e + rank).astype(jnp.int32)
    w_ref[...] = wcol

    brow = lax.broadcasted_iota(jnp.int32, (NB, E), 0) * B
    be = jnp.sum((brow >= offend.astype(jnp.int32)).astype(jnp.int32),
                 axis=1, keepdims=True)
    be_ref[...] = jnp.minimum(be, E - 1)
    xbf_ref[...] = x.astype(jnp.bfloat16)


def _router(x, w_gate, bias2d):
    return pl.pallas_call(
        _router_body,
        out_shape=[jax.ShapeDtypeStruct((NA, 1), jnp.int32),
                   jax.ShapeDtypeStruct((NA, 1), jnp.float32),
                   jax.ShapeDtypeStruct((NB, 1), jnp.int32),
                   jax.ShapeDtypeStruct((T, HID), jnp.bfloat16)],
    )(x, w_gate, bias2d)


def _dispatch_body(pos_hbm, w_hbm, rows_hbm, wslot_hbm, posv, wv, rowsv, wsv):
    cid = lax.axis_index("c")
    sid = lax.axis_index("s")

    @pl.when(jnp.logical_and(cid == 0, sid == 0))
    def _():
        pltpu.sync_copy(pos_hbm, posv)
        pltpu.sync_copy(w_hbm, wv)
        zi = jnp.zeros((16,), jnp.int32)
        zf = jnp.zeros((16,), jnp.float32)

        def zbody(i, carry):
            rowsv[pl.ds(i * 16, 16)] = zi
            wsv[pl.ds(i * 16, 16)] = zf
            return carry

        lax.fori_loop(0, NBB // 16, zbody, 0)
        lanes = lax.iota(jnp.int32, 16)

        def sbody(i, carry):
            idx = posv[pl.ds(i * 16, 16)]
            tok = jnp.bitwise_and(i * 16 + lanes, T - 1)
            plsc.store_scatter(rowsv, [idx], tok)
            plsc.store_scatter(wsv, [idx], wv[pl.ds(i * 16, 16)])
            return carry

        lax.fori_loop(0, NA // 16, sbody, 0)
        pltpu.sync_copy(rowsv, rows_hbm)
        pltpu.sync_copy(wsv, wslot_hbm)


def _dispatch(posf, wf):
    fn = pl.kernel(
        _dispatch_body,
        out_type=[jax.ShapeDtypeStruct((NBB,), jnp.int32),
                  jax.ShapeDtypeStruct((NBB,), jnp.float32)],
        mesh=plsc.VectorSubcoreMesh(core_axis_name="c", subcore_axis_name="s"),
        scratch_types=[pltpu.VMEM((NA,), jnp.int32),
                       pltpu.VMEM((NA,), jnp.float32),
                       pltpu.VMEM((NBB,), jnp.int32),
                       pltpu.VMEM((NBB,), jnp.float32)],
        compiler_params=pltpu.CompilerParams(needs_layout_passes=False),
    )
    return fn(posf, wf)


def _shared_body(x_ref, wsg_ref, wsu_ref, wsd_ref, out_ref):
    i = pl.program_id(0)
    x = x_ref[...]
    hg = _dotd(x, wsg_ref[...])
    hu = _dotd(x, wsu_ref[...])
    part = _dotd(hg * jax.nn.sigmoid(hg) * hu, wsd_ref[...])

    @pl.when(i == 0)
    def _():
        out_ref[...] = part

    @pl.when(i > 0)
    def _():
        out_ref[...] = out_ref[...] + part


def _shared(x, wsg, wsu, wsd):
    nst = SH_INTER // SH_TILE
    return pl.pallas_call(
        _shared_body,
        grid=(nst,),
        in_specs=[
            pl.BlockSpec((T, HID), lambda i: (0, 0)),
            pl.BlockSpec((HID, SH_TILE), lambda i: (0, i)),
            pl.BlockSpec((HID, SH_TILE), lambda i: (0, i)),
            pl.BlockSpec((SH_TILE, HID), lambda i: (i, 0)),
        ],
        out_specs=pl.BlockSpec((T, HID), lambda i: (0, 0)),
        out_shape=jax.ShapeDtypeStruct((T, HID), jnp.float32),
        compiler_params=pltpu.CompilerParams(
            dimension_semantics=("arbitrary",)),
    )(x, wsg, wsu, wsd)


def _moe_up_body(be_ref, rows_ref, xbf_ref, wg_ref, wu_ref, ws_ref, h_ref):
    rb = rows_ref[0]
    lane_t = lax.broadcasted_iota(jnp.int32, (B, T), 1)
    pb = (rb == lane_t).astype(jnp.bfloat16)
    xb = _dotd(pb, xbf_ref[...])
    hg = _dotd(xb, wg_ref[0])
    hu = _dotd(xb, wu_ref[0])
    h_ref[0] = hg * jax.nn.sigmoid(hg) * hu * ws_ref[0]


def _moe_up(bef, rows3, xbf, wg, wu, ws3):
    grid_spec = pltpu.PrefetchScalarGridSpec(
        num_scalar_prefetch=1,
        grid=(NB,),
        in_specs=[
            pl.BlockSpec((1, B, 1), lambda b, be: (b, 0, 0)),
            pl.BlockSpec((T, HID), lambda b, be: (0, 0)),
            pl.BlockSpec((1, HID, INTER), lambda b, be: (be[b], 0, 0)),
            pl.BlockSpec((1, HID, INTER), lambda b, be: (be[b], 0, 0)),
            pl.BlockSpec((1, B, 1), lambda b, be: (b, 0, 0)),
        ],
        out_specs=pl.BlockSpec((1, B, INTER), lambda b, be: (b, 0, 0)),
    )
    return pl.pallas_call(
        _moe_up_body,
        grid_spec=grid_spec,
        out_shape=jax.ShapeDtypeStruct((NB, B, INTER), jnp.float32),
        compiler_params=pltpu.CompilerParams(
            dimension_semantics=("arbitrary",),
            vmem_limit_bytes=60 * 1024 * 1024),
    )(bef, rows3, xbf, wg, wu, ws3)


def _moe_down_body(be_ref, h_ref, wd_ref, out_ref):
    out_ref[0] = _dotd(h_ref[0], wd_ref[0])


def _moe_down(bef, h3, wd):
    grid_spec = pltpu.PrefetchScalarGridSpec(
        num_scalar_prefetch=1,
        grid=(NB,),
        in_specs=[
            pl.BlockSpec((1, B, INTER), lambda b, be: (b, 0, 0)),
            pl.BlockSpec((1, INTER, HID), lambda b, be: (be[b], 0, 0)),
        ],
        out_specs=pl.BlockSpec((1, B, HID), lambda b, be: (b, 0, 0)),
    )
    return pl.pallas_call(
        _moe_down_body,
        grid_spec=grid_spec,
        out_shape=jax.ShapeDtypeStruct((NB, B, HID), jnp.float32),
        compiler_params=pltpu.CompilerParams(
            dimension_semantics=("arbitrary",),
            vmem_limit_bytes=60 * 1024 * 1024),
    )(bef, h3, wd)


_CCH = 8
_CN = (T // 32) // _CCH


def _combine_body(yg_hbm, pos_hbm, sh_hbm, out_hbm, p0v, p1v,
                  a0, a1, c0, c1, s0, s1, gs0, gs1, os0, os1):
    cid = lax.axis_index("c")
    sid = lax.axis_index("s")
    wid = sid * 2 + cid
    tbase = wid * (T // 32)
    abufs = (a0, a1)
    bbufs = (c0, c1)
    sbufs = (s0, s1)
    gsems = (gs0, gs1)
    osems = (os0, os1)

    pltpu.sync_copy(pos_hbm.at[pl.ds(tbase, T // 32)], p0v)
    pltpu.sync_copy(pos_hbm.at[pl.ds(T + tbase, T // 32)], p1v)

    def gstart(j):
        s = j % 2
        da = pltpu.async_copy(yg_hbm.at[p0v.at[pl.ds(j * _CCH, _CCH)]],
                              abufs[s], gsems[s])
        db = pltpu.async_copy(yg_hbm.at[p1v.at[pl.ds(j * _CCH, _CCH)]],
                              bbufs[s], gsems[s])
        dc = pltpu.async_copy(sh_hbm.at[pl.ds(tbase + j * _CCH, _CCH)],
                              sbufs[s], gsems[s])
        return (da, db, dc)

    gd = {0: gstart(0)}
    od = {}
    for j in range(_CN):
        if j + 1 < _CN:
            if j - 1 >= 0:
                od[j - 1].wait()
            gd[j + 1] = gstart(j + 1)
        for d in gd[j]:
            d.wait()
        s = j % 2
        a, c, sb = abufs[s], bbufs[s], sbufs[s]
        for r in range(_CCH):
            def cadd(k, carry2, r=r, a=a, c=c, sb=sb):
                sl = pl.ds(k * 16, 16)
                a[r, sl] = a[r, sl] + c[r, sl] + sb[r, sl]
                return carry2
            lax.fori_loop(0, HID // 16, cadd, 0)
        od[j] = pltpu.async_copy(
            a, out_hbm.at[pl.ds(tbase + j * _CCH, _CCH)], osems[s])
    od[_CN - 2].wait()
    od[_CN - 1].wait()


def _combine(yg, posf, shared):
    fn = pl.kernel(
        _combine_body,
        out_type=jax.ShapeDtypeStruct((T, HID), jnp.float32),
        mesh=plsc.VectorSubcoreMesh(core_axis_name="c", subcore_axis_name="s"),
        scratch_types=[pltpu.VMEM((T // 32,), jnp.int32),
                       pltpu.VMEM((T // 32,), jnp.int32),
                       pltpu.VMEM((_CCH, HID), jnp.float32),
                       pltpu.VMEM((_CCH, HID), jnp.float32),
                       pltpu.VMEM((_CCH, HID), jnp.float32),
                       pltpu.VMEM((_CCH, HID), jnp.float32),
                       pltpu.VMEM((_CCH, HID), jnp.float32),
                       pltpu.VMEM((_CCH, HID), jnp.float32),
                       pltpu.SemaphoreType.DMA, pltpu.SemaphoreType.DMA,
                       pltpu.SemaphoreType.DMA, pltpu.SemaphoreType.DMA],
    )
    return fn(yg, posf, shared)


def kernel(hidden_states, residual, layer_id, W_gate, bias_corr,
           Wg, Wu, Wd, Wsg, Wsu, Wsd):
    x = hidden_states
    pos, wcol, be, xbf = _router(x, W_gate, bias_corr.reshape(1, E))
    posf = pos.reshape(NA)
    wf = wcol.reshape(NA)
    bef = be.reshape(NB)
    rows, wslot = _dispatch(posf, wf)
    shared = _shared(x, Wsg, Wsu, Wsd)
    h3 = _moe_up(bef, rows.reshape(NB, B, 1), xbf, Wg, Wu,
                 wslot.reshape(NB, B, 1))
    yg = _moe_down(bef, h3, Wd)
    return _combine(yg.reshape(NBB, HID), posf, shared)


# speedup vs baseline: 2.3073x; 1.0091x over previous
import jax
import jax.numpy as jnp
from jax import lax
from jax.experimental import pallas as pl
from jax.experimental.pallas import tpu as pltpu
from jax.experimental.pallas import tpu_sc as plsc

T = 1024
HID = 2048
INTER = 1408
E = 16
GSZ = 4
RSF = 2.5
SH_INTER = 2816
SH_TILE = 256
NA = 2048
B = 128
NB = 32
NBB = NB * B
NEG = -1e9

_HI = lax.Precision.HIGHEST


def _dot(a, b):
    return lax.dot_general(a, b, (((1,), (0,)), ((), ())),
                           precision=_HI, preferred_element_type=jnp.float32)


def _dotd(a, b):
    return lax.dot_general(a, b, (((1,), (0,)), ((), ())),
                           preferred_element_type=jnp.float32)


def _router_body(x_ref, wg_ref, b_ref, pos_ref, w_ref, be_ref, xbf_ref):
    f32 = jnp.float32
    x = x_ref[...]
    logits = lax.dot_general(x, wg_ref[...], (((1,), (0,)), ((), ())),
                             preferred_element_type=jnp.float32)
    scores = jax.nn.sigmoid(logits)
    s4 = scores + b_ref[...]

    r16 = lax.broadcasted_iota(jnp.int32, (E, E), 0)
    c16 = lax.broadcasted_iota(jnp.int32, (E, E), 1)

    def roll(v, k):
        tgt = (c16 // GSZ) * GSZ + ((c16 % GSZ + k) % GSZ)
        return _dot(v, (r16 == tgt).astype(f32))

    u = jnp.maximum(s4 + roll(s4, 1), s4 + roll(s4, 2))
    v = jnp.maximum(u, roll(u, 1))
    gsc = jnp.maximum(v, roll(v, 2))

    lane = lax.broadcasted_iota(jnp.int32, (T, E), 1)
    gid = lane // GSZ
    gmax = jnp.max(gsc, axis=1, keepdims=True)
    i1g = jnp.min(jnp.where(gsc == gmax, gid, 99), axis=1, keepdims=True)
    g2s = jnp.where(gid == i1g, jnp.float32(-1e30), gsc)
    g2max = jnp.max(g2s, axis=1, keepdims=True)
    i2g = jnp.min(jnp.where(g2s == g2max, gid, 99), axis=1, keepdims=True)
    gmask = (gid == i1g) | (gid == i2g)

    msk = jnp.where(gmask, s4, NEG)
    m1 = jnp.max(msk, axis=1, keepdims=True)
    e1 = jnp.min(jnp.where(msk == m1, lane, 99), axis=1, keepdims=True)
    msk2 = jnp.where(lane == e1, NEG, msk)
    m2 = jnp.max(msk2, axis=1, keepdims=True)
    e2 = jnp.min(jnp.where(msk2 == m2, lane, 99), axis=1, keepdims=True)

    w1 = jnp.sum(jnp.where(lane == e1, scores, 0.0), axis=1, keepdims=True)
    w2 = jnp.sum(jnp.where(lane == e2, scores, 0.0), axis=1, keepdims=True)
    ws = w1 + w2 + 1e-20
    w1 = w1 / ws * RSF
    w2 = w2 / ws * RSF

    ecol = jnp.concatenate([e1, e2], axis=0)
    wcol = jnp.concatenate([w1, w2], axis=0)

    lane16 = lax.broadcasted_iota(jnp.int32, (NA, E), 1)
    oh = (ecol == lane16).astype(f32)
    cum = oh
    sh = 1
    while sh < NA:
        cum = cum + jnp.concatenate(
            [jnp.zeros((sh, E), f32), cum[:NA - sh, :]], axis=0)
        sh *= 2
    rank = jnp.sum(cum * oh, axis=1, keepdims=True) - 1.0
    cnt = cum[NA - 1:NA, :]
    cpad = ((cnt.astype(jnp.int32) + (B - 1)) // B) * B
    cpad_f = cpad.astype(f32)
    off = _dot(cpad_f, (r16 < c16).astype(f32))
    offend = off + cpad_f
    off_e = jnp.sum(oh * off, axis=1, keepdims=True)
    pos_ref[...] = (off_e + rank).astype(jnp.int32)
    w_ref[...] = wcol

    brow = lax.broadcasted_iota(jnp.int32, (NB, E), 0) * B
    be = jnp.sum((brow >= offend.astype(jnp.int32)).astype(jnp.int32),
                 axis=1, keepdims=True)
    be_ref[...] = jnp.minimum(be, E - 1)
    xbf_ref[...] = x.astype(jnp.bfloat16)


def _router(x, w_gate, bias2d):
    return pl.pallas_call(
        _router_body,
        out_shape=[jax.ShapeDtypeStruct((NA, 1), jnp.int32),
                   jax.ShapeDtypeStruct((NA, 1), jnp.float32),
                   jax.ShapeDtypeStruct((NB, 1), jnp.int32),
                   jax.ShapeDtypeStruct((T, HID), jnp.bfloat16)],
    )(x, w_gate, bias2d)


def _dispatch_body(pos_hbm, w_hbm, rows_hbm, wslot_hbm, posv, wv, rowsv, wsv):
    cid = lax.axis_index("c")
    sid = lax.axis_index("s")

    @pl.when(jnp.logical_and(cid == 0, sid == 0))
    def _():
        pltpu.sync_copy(pos_hbm, posv)
        pltpu.sync_copy(w_hbm, wv)
        zi = jnp.zeros((16,), jnp.int32)
        zf = jnp.zeros((16,), jnp.float32)

        def zbody(i, carry):
            rowsv[pl.ds(i * 16, 16)] = zi
            wsv[pl.ds(i * 16, 16)] = zf
            return carry

        lax.fori_loop(0, NBB // 16, zbody, 0)
        lanes = lax.iota(jnp.int32, 16)

        def sbody(i, carry):
            idx = posv[pl.ds(i * 16, 16)]
            tok = jnp.bitwise_and(i * 16 + lanes, T - 1)
            plsc.store_scatter(rowsv, [idx], tok)
            plsc.store_scatter(wsv, [idx], wv[pl.ds(i * 16, 16)])
            return carry

        lax.fori_loop(0, NA // 16, sbody, 0)
        pltpu.sync_copy(rowsv, rows_hbm)
        pltpu.sync_copy(wsv, wslot_hbm)


def _dispatch(posf, wf):
    fn = pl.kernel(
        _dispatch_body,
        out_type=[jax.ShapeDtypeStruct((NBB,), jnp.int32),
                  jax.ShapeDtypeStruct((NBB,), jnp.float32)],
        mesh=plsc.VectorSubcoreMesh(core_axis_name="c", subcore_axis_name="s"),
        scratch_types=[pltpu.VMEM((NA,), jnp.int32),
                       pltpu.VMEM((NA,), jnp.float32),
                       pltpu.VMEM((NBB,), jnp.int32),
                       pltpu.VMEM((NBB,), jnp.float32)],
        compiler_params=pltpu.CompilerParams(needs_layout_passes=False),
    )
    return fn(posf, wf)


def _shared_body(x_ref, wsg_ref, wsu_ref, wsd_ref, out_ref):
    i = pl.program_id(0)
    x = x_ref[...]
    hg = _dotd(x, wsg_ref[...])
    hu = _dotd(x, wsu_ref[...])
    part = _dotd(hg * jax.nn.sigmoid(hg) * hu, wsd_ref[...])

    @pl.when(i == 0)
    def _():
        out_ref[...] = part

    @pl.when(i > 0)
    def _():
        out_ref[...] = out_ref[...] + part


def _shared(x, wsg, wsu, wsd):
    nst = SH_INTER // SH_TILE
    return pl.pallas_call(
        _shared_body,
        grid=(nst,),
        in_specs=[
            pl.BlockSpec((T, HID), lambda i: (0, 0)),
            pl.BlockSpec((HID, SH_TILE), lambda i: (0, i)),
            pl.BlockSpec((HID, SH_TILE), lambda i: (0, i)),
            pl.BlockSpec((SH_TILE, HID), lambda i: (i, 0)),
        ],
        out_specs=pl.BlockSpec((T, HID), lambda i: (0, 0)),
        out_shape=jax.ShapeDtypeStruct((T, HID), jnp.float32),
        compiler_params=pltpu.CompilerParams(
            dimension_semantics=("arbitrary",)),
    )(x, wsg, wsu, wsd)


def _moe_up_body(be_ref, rows_ref, xbf_ref, wg_ref, wu_ref, ws_ref, h_ref):
    rb = rows_ref[0]
    lane_t = lax.broadcasted_iota(jnp.int32, (B, T), 1)
    pb = (rb == lane_t).astype(jnp.bfloat16)
    xb = _dotd(pb, xbf_ref[...])
    hg = _dotd(xb, wg_ref[0])
    hu = _dotd(xb, wu_ref[0])
    h_ref[0] = (hg * jax.nn.sigmoid(hg) * hu * ws_ref[0]).astype(jnp.bfloat16)


def _moe_up(bef, rows3, xbf, wg, wu, ws3):
    grid_spec = pltpu.PrefetchScalarGridSpec(
        num_scalar_prefetch=1,
        grid=(NB,),
        in_specs=[
            pl.BlockSpec((1, B, 1), lambda b, be: (b, 0, 0)),
            pl.BlockSpec((T, HID), lambda b, be: (0, 0)),
            pl.BlockSpec((1, HID, INTER), lambda b, be: (be[b], 0, 0)),
            pl.BlockSpec((1, HID, INTER), lambda b, be: (be[b], 0, 0)),
            pl.BlockSpec((1, B, 1), lambda b, be: (b, 0, 0)),
        ],
        out_specs=pl.BlockSpec((1, B, INTER), lambda b, be: (b, 0, 0)),
    )
    return pl.pallas_call(
        _moe_up_body,
        grid_spec=grid_spec,
        out_shape=jax.ShapeDtypeStruct((NB, B, INTER), jnp.bfloat16),
        compiler_params=pltpu.CompilerParams(
            dimension_semantics=("arbitrary",),
            vmem_limit_bytes=60 * 1024 * 1024),
    )(bef, rows3, xbf, wg, wu, ws3)


def _moe_down_body(be_ref, h_ref, wd_ref, out_ref):
    out_ref[0] = _dotd(h_ref[0].astype(jnp.float32), wd_ref[0])


def _moe_down(bef, h3, wd):
    grid_spec = pltpu.PrefetchScalarGridSpec(
        num_scalar_prefetch=1,
        grid=(NB,),
        in_specs=[
            pl.BlockSpec((1, B, INTER), lambda b, be: (b, 0, 0)),
            pl.BlockSpec((1, INTER, HID), lambda b, be: (be[b], 0, 0)),
        ],
        out_specs=pl.BlockSpec((1, B, HID), lambda b, be: (b, 0, 0)),
    )
    return pl.pallas_call(
        _moe_down_body,
        grid_spec=grid_spec,
        out_shape=jax.ShapeDtypeStruct((NB, B, HID), jnp.float32),
        compiler_params=pltpu.CompilerParams(
            dimension_semantics=("arbitrary",),
            vmem_limit_bytes=60 * 1024 * 1024),
    )(bef, h3, wd)


_CCH = 8
_CN = (T // 32) // _CCH


def _combine_body(yg_hbm, pos_hbm, sh_hbm, out_hbm, p0v, p1v,
                  a0, a1, c0, c1, s0, s1, gs0, gs1, os0, os1):
    cid = lax.axis_index("c")
    sid = lax.axis_index("s")
    wid = sid * 2 + cid
    tbase = wid * (T // 32)
    abufs = (a0, a1)
    bbufs = (c0, c1)
    sbufs = (s0, s1)
    gsems = (gs0, gs1)
    osems = (os0, os1)

    pltpu.sync_copy(pos_hbm.at[pl.ds(tbase, T // 32)], p0v)
    pltpu.sync_copy(pos_hbm.at[pl.ds(T + tbase, T // 32)], p1v)

    def gstart(j):
        s = j % 2
        da = pltpu.async_copy(yg_hbm.at[p0v.at[pl.ds(j * _CCH, _CCH)]],
                              abufs[s], gsems[s])
        db = pltpu.async_copy(yg_hbm.at[p1v.at[pl.ds(j * _CCH, _CCH)]],
                              bbufs[s], gsems[s])
        dc = pltpu.async_copy(sh_hbm.at[pl.ds(tbase + j * _CCH, _CCH)],
                              sbufs[s], gsems[s])
        return (da, db, dc)

    gd = {0: gstart(0)}
    od = {}
    for j in range(_CN):
        if j + 1 < _CN:
            if j - 1 >= 0:
                od[j - 1].wait()
            gd[j + 1] = gstart(j + 1)
        for d in gd[j]:
            d.wait()
        s = j % 2
        a, c, sb = abufs[s], bbufs[s], sbufs[s]
        for r in range(_CCH):
            def cadd(k, carry2, r=r, a=a, c=c, sb=sb):
                sl = pl.ds(k * 16, 16)
                a[r, sl] = a[r, sl] + c[r, sl] + sb[r, sl]
                return carry2
            lax.fori_loop(0, HID // 16, cadd, 0)
        od[j] = pltpu.async_copy(
            a, out_hbm.at[pl.ds(tbase + j * _CCH, _CCH)], osems[s])
    od[_CN - 2].wait()
    od[_CN - 1].wait()


def _combine(yg, posf, shared):
    fn = pl.kernel(
        _combine_body,
        out_type=jax.ShapeDtypeStruct((T, HID), jnp.float32),
        mesh=plsc.VectorSubcoreMesh(core_axis_name="c", subcore_axis_name="s"),
        scratch_types=[pltpu.VMEM((T // 32,), jnp.int32),
                       pltpu.VMEM((T // 32,), jnp.int32),
                       pltpu.VMEM((_CCH, HID), jnp.float32),
                       pltpu.VMEM((_CCH, HID), jnp.float32),
                       pltpu.VMEM((_CCH, HID), jnp.float32),
                       pltpu.VMEM((_CCH, HID), jnp.float32),
                       pltpu.VMEM((_CCH, HID), jnp.float32),
                       pltpu.VMEM((_CCH, HID), jnp.float32),
                       pltpu.SemaphoreType.DMA, pltpu.SemaphoreType.DMA,
                       pltpu.SemaphoreType.DMA, pltpu.SemaphoreType.DMA],
    )
    return fn(yg, posf, shared)


def kernel(hidden_states, residual, layer_id, W_gate, bias_corr,
           Wg, Wu, Wd, Wsg, Wsu, Wsd):
    x = hidden_states
    pos, wcol, be, xbf = _router(x, W_gate, bias_corr.reshape(1, E))
    posf = pos.reshape(NA)
    wf = wcol.reshape(NA)
    bef = be.reshape(NB)
    rows, wslot = _dispatch(posf, wf)
    shared = _shared(x, Wsg, Wsu, Wsd)
    h3 = _moe_up(bef, rows.reshape(NB, B, 1), xbf, Wg, Wu,
                 wslot.reshape(NB, B, 1))
    yg = _moe_down(bef, h3, Wd)
    return _combine(yg.reshape(NBB, HID), posf, shared)


# speedup vs baseline: 2.3713x; 1.0277x over previous
import jax
import jax.numpy as jnp
from jax import lax
from jax.experimental import pallas as pl
from jax.experimental.pallas import tpu as pltpu
from jax.experimental.pallas import tpu_sc as plsc

T = 1024
HID = 2048
INTER = 1408
E = 16
GSZ = 4
RSF = 2.5
SH_INTER = 2816
SH_TILE = 256
NA = 2048
B = 128
NB = 32
NBB = NB * B
NEG = -1e9

_HI = lax.Precision.HIGHEST


def _dot(a, b):
    return lax.dot_general(a, b, (((1,), (0,)), ((), ())),
                           precision=_HI, preferred_element_type=jnp.float32)


def _dotd(a, b):
    return lax.dot_general(a, b, (((1,), (0,)), ((), ())),
                           preferred_element_type=jnp.float32)


def _router_body(x_ref, wg_ref, b_ref, pos_ref, w_ref, be_ref, xbf_ref):
    f32 = jnp.float32
    x = x_ref[...]
    logits = lax.dot_general(x, wg_ref[...], (((1,), (0,)), ((), ())),
                             preferred_element_type=jnp.float32)
    scores = jax.nn.sigmoid(logits)
    s4 = scores + b_ref[...]

    r16 = lax.broadcasted_iota(jnp.int32, (E, E), 0)
    c16 = lax.broadcasted_iota(jnp.int32, (E, E), 1)

    def roll(v, k):
        tgt = (c16 // GSZ) * GSZ + ((c16 % GSZ + k) % GSZ)
        return _dot(v, (r16 == tgt).astype(f32))

    u = jnp.maximum(s4 + roll(s4, 1), s4 + roll(s4, 2))
    v = jnp.maximum(u, roll(u, 1))
    gsc = jnp.maximum(v, roll(v, 2))

    lane = lax.broadcasted_iota(jnp.int32, (T, E), 1)
    gid = lane // GSZ
    gmax = jnp.max(gsc, axis=1, keepdims=True)
    i1g = jnp.min(jnp.where(gsc == gmax, gid, 99), axis=1, keepdims=True)
    g2s = jnp.where(gid == i1g, jnp.float32(-1e30), gsc)
    g2max = jnp.max(g2s, axis=1, keepdims=True)
    i2g = jnp.min(jnp.where(g2s == g2max, gid, 99), axis=1, keepdims=True)
    gmask = (gid == i1g) | (gid == i2g)

    msk = jnp.where(gmask, s4, NEG)
    m1 = jnp.max(msk, axis=1, keepdims=True)
    e1 = jnp.min(jnp.where(msk == m1, lane, 99), axis=1, keepdims=True)
    msk2 = jnp.where(lane == e1, NEG, msk)
    m2 = jnp.max(msk2, axis=1, keepdims=True)
    e2 = jnp.min(jnp.where(msk2 == m2, lane, 99), axis=1, keepdims=True)

    w1 = jnp.sum(jnp.where(lane == e1, scores, 0.0), axis=1, keepdims=True)
    w2 = jnp.sum(jnp.where(lane == e2, scores, 0.0), axis=1, keepdims=True)
    ws = w1 + w2 + 1e-20
    w1 = w1 / ws * RSF
    w2 = w2 / ws * RSF

    ecol = jnp.concatenate([e1, e2], axis=0)
    wcol = jnp.concatenate([w1, w2], axis=0)

    lane16 = lax.broadcasted_iota(jnp.int32, (NA, E), 1)
    oh = (ecol == lane16).astype(f32)
    cum = oh
    sh = 1
    while sh < NA:
        cum = cum + jnp.concatenate(
            [jnp.zeros((sh, E), f32), cum[:NA - sh, :]], axis=0)
        sh *= 2
    rank = jnp.sum(cum * oh, axis=1, keepdims=True) - 1.0
    cnt = cum[NA - 1:NA, :]
    cpad = ((cnt.astype(jnp.int32) + (B - 1)) // B) * B
    cpad_f = cpad.astype(f32)
    off = _dot(cpad_f, (r16 < c16).astype(f32))
    offend = off + cpad_f
    off_e = jnp.sum(oh * off, axis=1, keepdims=True)
    pos_ref[...] = (off_e + rank).astype(jnp.int32)
    w_ref[...] = wcol

    brow = lax.broadcasted_iota(jnp.int32, (NB, E), 0) * B
    be = jnp.sum((brow >= offend.astype(jnp.int32)).astype(jnp.int32),
                 axis=1, keepdims=True)
    be_ref[...] = jnp.minimum(be, E - 1)
    xbf_ref[...] = x.astype(jnp.bfloat16)


def _router(x, w_gate, bias2d):
    return pl.pallas_call(
        _router_body,
        out_shape=[jax.ShapeDtypeStruct((NA, 1), jnp.int32),
                   jax.ShapeDtypeStruct((NA, 1), jnp.float32),
                   jax.ShapeDtypeStruct((NB, 1), jnp.int32),
                   jax.ShapeDtypeStruct((T, HID), jnp.bfloat16)],
    )(x, w_gate, bias2d)


def _dispatch_body(pos_hbm, w_hbm, rows_hbm, wslot_hbm, posv, wv, rowsv, wsv):
    cid = lax.axis_index("c")
    sid = lax.axis_index("s")

    @pl.when(jnp.logical_and(cid == 0, sid == 0))
    def _():
        pltpu.sync_copy(pos_hbm, posv)
        pltpu.sync_copy(w_hbm, wv)
        zi = jnp.zeros((16,), jnp.int32)
        zf = jnp.zeros((16,), jnp.float32)

        def zbody(i, carry):
            rowsv[pl.ds(i * 16, 16)] = zi
            wsv[pl.ds(i * 16, 16)] = zf
            return carry

        lax.fori_loop(0, NBB // 16, zbody, 0)
        lanes = lax.iota(jnp.int32, 16)

        def sbody(i, carry):
            idx = posv[pl.ds(i * 16, 16)]
            tok = jnp.bitwise_and(i * 16 + lanes, T - 1)
            plsc.store_scatter(rowsv, [idx], tok)
            plsc.store_scatter(wsv, [idx], wv[pl.ds(i * 16, 16)])
            return carry

        lax.fori_loop(0, NA // 16, sbody, 0)
        pltpu.sync_copy(rowsv, rows_hbm)
        pltpu.sync_copy(wsv, wslot_hbm)


def _dispatch(posf, wf):
    fn = pl.kernel(
        _dispatch_body,
        out_type=[jax.ShapeDtypeStruct((NBB,), jnp.int32),
                  jax.ShapeDtypeStruct((NBB,), jnp.float32)],
        mesh=plsc.VectorSubcoreMesh(core_axis_name="c", subcore_axis_name="s"),
        scratch_types=[pltpu.VMEM((NA,), jnp.int32),
                       pltpu.VMEM((NA,), jnp.float32),
                       pltpu.VMEM((NBB,), jnp.int32),
                       pltpu.VMEM((NBB,), jnp.float32)],
        compiler_params=pltpu.CompilerParams(needs_layout_passes=False),
    )
    return fn(posf, wf)


def _shared_body(x_ref, wsg_ref, wsu_ref, wsd_ref, out_ref):
    i = pl.program_id(0)
    x = x_ref[...]
    hg = _dotd(x, wsg_ref[...])
    hu = _dotd(x, wsu_ref[...])
    part = _dotd(hg * jax.nn.sigmoid(hg) * hu, wsd_ref[...])

    @pl.when(i == 0)
    def _():
        out_ref[...] = part

    @pl.when(i > 0)
    def _():
        out_ref[...] = out_ref[...] + part


def _shared(x, wsg, wsu, wsd):
    nst = SH_INTER // SH_TILE
    return pl.pallas_call(
        _shared_body,
        grid=(nst,),
        in_specs=[
            pl.BlockSpec((T, HID), lambda i: (0, 0)),
            pl.BlockSpec((HID, SH_TILE), lambda i: (0, i)),
            pl.BlockSpec((HID, SH_TILE), lambda i: (0, i)),
            pl.BlockSpec((SH_TILE, HID), lambda i: (i, 0)),
        ],
        out_specs=pl.BlockSpec((T, HID), lambda i: (0, 0)),
        out_shape=jax.ShapeDtypeStruct((T, HID), jnp.float32),
        compiler_params=pltpu.CompilerParams(
            dimension_semantics=("arbitrary",)),
    )(x, wsg, wsu, wsd)


def _moe_up_body(be_ref, rows_ref, xbf_ref, wg_ref, wu_ref, ws_ref, h_ref):
    rb = rows_ref[0]
    lane_t = lax.broadcasted_iota(jnp.int32, (B, T), 1)
    pb = (rb == lane_t).astype(jnp.bfloat16)
    xb = _dotd(pb, xbf_ref[...])
    hg = _dotd(xb, wg_ref[0])
    hu = _dotd(xb, wu_ref[0])
    h_ref[0] = (hg * jax.nn.sigmoid(hg) * hu * ws_ref[0]).astype(jnp.bfloat16)


def _moe_up(bef, rows3, xbf, wg, wu, ws3):
    grid_spec = pltpu.PrefetchScalarGridSpec(
        num_scalar_prefetch=1,
        grid=(NB,),
        in_specs=[
            pl.BlockSpec((1, B, 1), lambda b, be: (b, 0, 0)),
            pl.BlockSpec((T, HID), lambda b, be: (0, 0)),
            pl.BlockSpec((1, HID, INTER), lambda b, be: (be[b], 0, 0)),
            pl.BlockSpec((1, HID, INTER), lambda b, be: (be[b], 0, 0)),
            pl.BlockSpec((1, B, 1), lambda b, be: (b, 0, 0)),
        ],
        out_specs=pl.BlockSpec((1, B, INTER), lambda b, be: (b, 0, 0)),
    )
    return pl.pallas_call(
        _moe_up_body,
        grid_spec=grid_spec,
        out_shape=jax.ShapeDtypeStruct((NB, B, INTER), jnp.bfloat16),
        compiler_params=pltpu.CompilerParams(
            dimension_semantics=("arbitrary",),
            vmem_limit_bytes=60 * 1024 * 1024),
    )(bef, rows3, xbf, wg, wu, ws3)


def _moe_down_body(be_ref, h_ref, wd_ref, out_ref):
    out_ref[0] = _dotd(h_ref[0].astype(jnp.float32),
                       wd_ref[0]).astype(jnp.bfloat16)


def _moe_down(bef, h3, wd):
    grid_spec = pltpu.PrefetchScalarGridSpec(
        num_scalar_prefetch=1,
        grid=(NB,),
        in_specs=[
            pl.BlockSpec((1, B, INTER), lambda b, be: (b, 0, 0)),
            pl.BlockSpec((1, INTER, HID), lambda b, be: (be[b], 0, 0)),
        ],
        out_specs=pl.BlockSpec((1, B, HID), lambda b, be: (b, 0, 0)),
    )
    return pl.pallas_call(
        _moe_down_body,
        grid_spec=grid_spec,
        out_shape=jax.ShapeDtypeStruct((NB, B, HID), jnp.bfloat16),
        compiler_params=pltpu.CompilerParams(
            dimension_semantics=("arbitrary",),
            vmem_limit_bytes=60 * 1024 * 1024),
    )(bef, h3, wd)


_CW = 512


def _combine_body(pos_ref, yg_ref, sh_ref, out_ref, p_ref):
    i = pl.program_id(0)

    @pl.when(i == 0)
    def _():
        posc = pos_ref[...]
        lane = lax.broadcasted_iota(jnp.int32, (T, NBB), 1)
        p_ref[...] = ((lane == posc[:T, :]) |
                      (lane == posc[T:, :])).astype(jnp.bfloat16)

    out_ref[...] = _dotd(p_ref[...], yg_ref[...]) + sh_ref[...]


def _combine(pos, yg, shared):
    return pl.pallas_call(
        _combine_body,
        grid=(HID // _CW,),
        in_specs=[
            pl.BlockSpec((NA, 1), lambda i: (0, 0)),
            pl.BlockSpec((NBB, _CW), lambda i: (0, i)),
            pl.BlockSpec((T, _CW), lambda i: (0, i)),
        ],
        out_specs=pl.BlockSpec((T, _CW), lambda i: (0, i)),
        out_shape=jax.ShapeDtypeStruct((T, HID), jnp.float32),
        scratch_shapes=[pltpu.VMEM((T, NBB), jnp.bfloat16)],
        compiler_params=pltpu.CompilerParams(
            dimension_semantics=("arbitrary",)),
    )(pos, yg, shared)


def kernel(hidden_states, residual, layer_id, W_gate, bias_corr,
           Wg, Wu, Wd, Wsg, Wsu, Wsd):
    x = hidden_states
    pos, wcol, be, xbf = _router(x, W_gate, bias_corr.reshape(1, E))
    posf = pos.reshape(NA)
    wf = wcol.reshape(NA)
    bef = be.reshape(NB)
    rows, wslot = _dispatch(posf, wf)
    shared = _shared(x, Wsg, Wsu, Wsd)
    h3 = _moe_up(bef, rows.reshape(NB, B, 1), xbf, Wg, Wu,
                 wslot.reshape(NB, B, 1))
    yg = _moe_down(bef, h3, Wd)
    return _combine(pos, yg.reshape(NBB, HID), shared)
